# baseline (speedup 1.0000x reference)
import functools

import jax
import jax.numpy as jnp
from jax import lax
from jax.experimental import pallas as pl
from jax.experimental.pallas import tpu as pltpu
from jax.experimental.pallas import tpu_sc as plsc

NUM_EMB = 1000000
DIM = 64
B_SEQ = 16384
S_SEQ = 50
B_TOTAL = B_SEQ * S_SEQ

_info = plsc.get_sparse_core_info()
NC, NS = _info.num_cores, _info.num_subcores
NW = NC * NS
CHUNK = 128
NPAIR = B_TOTAL // CHUNK
PPW = NPAIR // NW
TR = DIM // 8
TILE = 8 * CHUNK

_mesh = plsc.VectorSubcoreMesh(core_axis_name="c", subcore_axis_name="s")


@functools.partial(
    pl.kernel,
    mesh=_mesh,
    out_type=jax.ShapeDtypeStruct((S_SEQ, TR, B_SEQ // CHUNK, TILE), jnp.float32),
    scratch_types=[
        pltpu.VMEM((PPW, CHUNK), jnp.int32),
        pltpu.VMEM((2, CHUNK, DIM), jnp.float32),
        pltpu.VMEM((2, TR * TILE), jnp.float32),
        pltpu.SemaphoreType.DMA((2,)),
        pltpu.SemaphoreType.DMA((2,)),
    ],
    compiler_params=pltpu.CompilerParams(use_tc_tiling_on_sc=False,
                                         needs_layout_passes=False),
)
def _emb_lookup(table_hbm, idx_hbm, raw_hbm, idx_all, stag, tbuf, gsem, ssem):
    wid = lax.axis_index("s") * NC + lax.axis_index("c")
    base_p = wid * PPW

    pltpu.sync_copy(idx_hbm.at[pl.ds(base_p, PPW)], idx_all)

    lane = lax.iota(jnp.int32, 16)
    p128 = lane * CHUNK

    def fire(k, h):
        pltpu.async_copy(table_hbm.at[idx_all.at[k]], stag.at[h], gsem.at[h])

    def gwait(k, h):
        pltpu.make_async_copy(table_hbm.at[idx_all.at[k]], stag.at[h],
                              gsem.at[h]).wait()

    def transpose(h):
        for b in range(CHUNK):
            for d0 in range(0, DIM, 16):
                v = stag[h, b, pl.ds(d0, 16)]
                plsc.store_scatter(tbuf.at[h], [p128 + (d0 * CHUNK + b)], v)

    def store_descs(k, h):
        p = base_p + k
        s = p // (B_SEQ // CHUNK)
        tc = lax.rem(p, B_SEQ // CHUNK)
        return [
            pltpu.make_async_copy(tbuf.at[h, pl.ds(tr * TILE, TILE)],
                                  raw_hbm.at[s, tr, tc], ssem.at[h])
            for tr in range(TR)
        ]

    fire(0, 0)

    def loop_body(t, carry):
        for h in (0, 1):
            k = 2 * t + h
            gwait(k, h)
            if h == 0:
                fire(k + 1, 1)
            else:
                @pl.when(t < PPW // 2 - 1)
                def _():
                    fire(k + 1, 0)

            @pl.when(t >= 1)
            def _():
                for d in store_descs(k - 2, h):
                    d.wait()

            transpose(h)
            for d in store_descs(k, h):
                d.start()
        return carry

    lax.fori_loop(0, PPW // 2, loop_body, 0)
    for h in (0, 1):
        for d in store_descs(PPW - 2 + h, h):
            d.wait()


def kernel(x, weight):
    xt = x.T.reshape(NPAIR, CHUNK).astype(jnp.int32)
    raw = _emb_lookup(weight, xt)
    out = raw.reshape(S_SEQ, TR, B_SEQ // CHUNK, 8, CHUNK)
    out = out.transpose(2, 4, 0, 1, 3).reshape(B_SEQ, S_SEQ, DIM)
    return out


# speedup vs baseline: 1.2311x; 1.2311x over previous
import functools

import jax
import jax.numpy as jnp
from jax import lax
from jax.experimental import pallas as pl
from jax.experimental.pallas import tpu as pltpu
from jax.experimental.pallas import tpu_sc as plsc

NUM_EMB = 1000000
DIM = 64
B_SEQ = 16384
S_SEQ = 50
B_TOTAL = B_SEQ * S_SEQ

_info = plsc.get_sparse_core_info()
NC, NS = _info.num_cores, _info.num_subcores
NW = NC * NS
CHUNK = 128
NPAIR = B_TOTAL // CHUNK
PPW = NPAIR // NW
TR = DIM // 8
TILE = 8 * CHUNK

_mesh = plsc.VectorSubcoreMesh(core_axis_name="c", subcore_axis_name="s")


@functools.partial(
    pl.kernel,
    mesh=_mesh,
    out_type=jax.ShapeDtypeStruct((S_SEQ, TR, B_SEQ // CHUNK, TILE), jnp.float32),
    scratch_types=[
        pltpu.VMEM((PPW, CHUNK), jnp.int32),
        pltpu.VMEM((2, CHUNK, DIM), jnp.float32),
        pltpu.VMEM((2, TR * TILE), jnp.float32),
        pltpu.SemaphoreType.DMA((2,)),
        pltpu.SemaphoreType.DMA((2,)),
    ],
    compiler_params=pltpu.CompilerParams(use_tc_tiling_on_sc=False,
                                         needs_layout_passes=False),
)
def _emb_lookup(table_hbm, idx_hbm, raw_hbm, idx_all, stag, tbuf, gsem, ssem):
    wid = lax.axis_index("s") * NC + lax.axis_index("c")
    base_p = wid * PPW

    pltpu.sync_copy(idx_hbm.at[pl.ds(base_p, PPW)], idx_all)

    lane = lax.iota(jnp.int32, 16)
    p128 = lane * CHUNK
    pvec = [p128 + (d0 * CHUNK) for d0 in range(0, DIM, 16)]

    def fire(k, h):
        pltpu.async_copy(table_hbm.at[idx_all.at[k]], stag.at[h], gsem.at[h])

    def gwait(k, h):
        pltpu.make_async_copy(table_hbm.at[idx_all.at[k]], stag.at[h],
                              gsem.at[h]).wait()

    def transpose(h):
        @plsc.parallel_loop(0, CHUNK, 1, unroll=8)
        def _(b):
            for i, d0 in enumerate(range(0, DIM, 16)):
                v = stag[h, b, pl.ds(d0, 16)]
                plsc.store_scatter(tbuf.at[h], [pvec[i] + b], v)

    def store_descs(k, h):
        p = base_p + k
        s = p // (B_SEQ // CHUNK)
        tc = lax.rem(p, B_SEQ // CHUNK)
        return [
            pltpu.make_async_copy(tbuf.at[h, pl.ds(tr * TILE, TILE)],
                                  raw_hbm.at[s, tr, tc], ssem.at[h])
            for tr in range(TR)
        ]

    fire(0, 0)

    def loop_body(t, carry):
        for h in (0, 1):
            k = 2 * t + h
            gwait(k, h)
            if h == 0:
                fire(k + 1, 1)
            else:
                @pl.when(t < PPW // 2 - 1)
                def _():
                    fire(k + 1, 0)

            @pl.when(t >= 1)
            def _():
                for d in store_descs(k - 2, h):
                    d.wait()

            transpose(h)
            for d in store_descs(k, h):
                d.start()
        return carry

    lax.fori_loop(0, PPW // 2, loop_body, 0)
    for h in (0, 1):
        for d in store_descs(PPW - 2 + h, h):
            d.wait()


def kernel(x, weight):
    xt = x.T.reshape(NPAIR, CHUNK).astype(jnp.int32)
    raw = _emb_lookup(weight, xt)
    out = raw.reshape(S_SEQ, TR, B_SEQ // CHUNK, 8, CHUNK)
    out = out.transpose(2, 4, 0, 1, 3).reshape(B_SEQ, S_SEQ, DIM)
    return out


# speedup vs baseline: 1.7844x; 1.4494x over previous
import functools

import jax
import jax.numpy as jnp
from jax import lax
from jax.experimental import pallas as pl
from jax.experimental.pallas import tpu as pltpu
from jax.experimental.pallas import tpu_sc as plsc

NUM_EMB = 1000000
DIM = 64
B_SEQ = 16384
S_SEQ = 50
B_TOTAL = B_SEQ * S_SEQ

_info = plsc.get_sparse_core_info()
NC, NS = _info.num_cores, _info.num_subcores
NW = NC * NS
CHUNK = 128
NPAIR = B_TOTAL // CHUNK
PPW = NPAIR // NW
TR = DIM // 8
TILE = 8 * CHUNK

_mesh = plsc.VectorSubcoreMesh(core_axis_name="c", subcore_axis_name="s")


@functools.partial(
    pl.kernel,
    mesh=_mesh,
    out_type=jax.ShapeDtypeStruct((S_SEQ, TR, B_SEQ // CHUNK, TILE), jnp.float32),
    scratch_types=[
        pltpu.VMEM((PPW, CHUNK), jnp.int32),
        pltpu.VMEM((2, CHUNK, DIM), jnp.float32),
        pltpu.VMEM((2, TR * TILE), jnp.float32),
        pltpu.SemaphoreType.DMA((2,)),
        pltpu.SemaphoreType.DMA((2,)),
    ],
    compiler_params=pltpu.CompilerParams(use_tc_tiling_on_sc=False,
                                         needs_layout_passes=False),
)
def _emb_lookup(table_hbm, idx_hbm, raw_hbm, idx_all, stag, tbuf, gsem, ssem):
    wid = lax.axis_index("s") * NC + lax.axis_index("c")
    base_p = wid * PPW

    pltpu.sync_copy(idx_hbm.at[pl.ds(base_p, PPW)], idx_all)

    lane = lax.iota(jnp.int32, 16)
    cdiag = [jnp.where(lane + j > 15, lane + j - 16, lane + j)
             for j in range(16)]
    wdiag = [cdiag[j] * CHUNK + lane for j in range(16)]

    def fire(k, h):
        pltpu.async_copy(table_hbm.at[idx_all.at[k]], stag.at[h], gsem.at[h])

    def gwait(k, h):
        pltpu.make_async_copy(table_hbm.at[idx_all.at[k]], stag.at[h],
                              gsem.at[h]).wait()

    def transpose(h):
        @plsc.parallel_loop(0, CHUNK, 16, unroll=2)
        def _(b0):
            rows = lane + b0
            for d0 in range(0, DIM, 16):
                for j in range(16):
                    v = plsc.load_gather(stag.at[h], [rows, cdiag[j] + d0])
                    plsc.store_scatter(tbuf.at[h],
                                       [wdiag[j] + (d0 * CHUNK + b0)], v)

    def store_descs(k, h):
        p = base_p + k
        s = p // (B_SEQ // CHUNK)
        tc = lax.rem(p, B_SEQ // CHUNK)
        return [
            pltpu.make_async_copy(tbuf.at[h, pl.ds(tr * TILE, TILE)],
                                  raw_hbm.at[s, tr, tc], ssem.at[h])
            for tr in range(TR)
        ]

    fire(0, 0)

    def loop_body(t, carry):
        for h in (0, 1):
            k = 2 * t + h
            gwait(k, h)
            if h == 0:
                fire(k + 1, 1)
            else:
                @pl.when(t < PPW // 2 - 1)
                def _():
                    fire(k + 1, 0)

            @pl.when(t >= 1)
            def _():
                for d in store_descs(k - 2, h):
                    d.wait()

            transpose(h)
            for d in store_descs(k, h):
                d.start()
        return carry

    lax.fori_loop(0, PPW // 2, loop_body, 0)
    for h in (0, 1):
        for d in store_descs(PPW - 2 + h, h):
            d.wait()


def kernel(x, weight):
    xt = x.T.reshape(NPAIR, CHUNK).astype(jnp.int32)
    raw = _emb_lookup(weight, xt)
    out = raw.reshape(S_SEQ, TR, B_SEQ // CHUNK, 8, CHUNK)
    out = out.transpose(2, 4, 0, 1, 3).reshape(B_SEQ, S_SEQ, DIM)
    return out


# speedup vs baseline: 2.1073x; 1.1810x over previous
import functools

import jax
import jax.numpy as jnp
from jax import lax
from jax.experimental import pallas as pl
from jax.experimental.pallas import tpu as pltpu
from jax.experimental.pallas import tpu_sc as plsc

NUM_EMB = 1000000
DIM = 64
B_SEQ = 16384
S_SEQ = 50
B_TOTAL = B_SEQ * S_SEQ

_info = plsc.get_sparse_core_info()
NC, NS = _info.num_cores, _info.num_subcores
NW = NC * NS
CHUNK = 128
NPAIR = B_TOTAL // CHUNK
PPW = NPAIR // NW
TR = DIM // 8
TILE = 8 * CHUNK

_mesh = plsc.VectorSubcoreMesh(core_axis_name="c", subcore_axis_name="s")

NSLAB = NUM_EMB // CHUNK
SPW_CEIL = NSLAB // NW + 1


@functools.partial(
    pl.kernel,
    mesh=_mesh,
    out_type=jax.ShapeDtypeStruct((NUM_EMB * DIM,), jnp.float32),
    scratch_types=[
        pltpu.VMEM((DIM, CHUNK), jnp.float32),
        pltpu.VMEM((DIM, CHUNK), jnp.float32),
        pltpu.VMEM((CHUNK * DIM,), jnp.float32),
        pltpu.VMEM((CHUNK * DIM,), jnp.float32),
        pltpu.SemaphoreType.DMA,
        pltpu.SemaphoreType.DMA,
        pltpu.SemaphoreType.DMA,
        pltpu.SemaphoreType.DMA,
    ],
    compiler_params=pltpu.CompilerParams(use_tc_tiling_on_sc=True,
                                         needs_layout_passes=False),
)
def _table_transpose(wt_hbm, wrow_hbm, sbuf0, sbuf1, obuf0, obuf1,
                     isem0, isem1, osem0, osem1):
    wid = lax.axis_index("s") * NC + lax.axis_index("c")
    sbuf = (sbuf0, sbuf1)
    obuf = (obuf0, obuf1)
    isem = (isem0, isem1)
    osem = (osem0, osem1)

    lane = lax.iota(jnp.int32, 16)
    cdiag = [jnp.where(lane + j > 15, lane + j - 16, lane + j)
             for j in range(16)]
    wdiag = [cdiag[j] * DIM + lane for j in range(16)]

    def slab(t):
        return wid + NW * t

    def fire_in(t, h):
        pltpu.async_copy(wt_hbm.at[:, pl.ds(slab(t) * CHUNK, CHUNK)],
                         sbuf[h], isem[h])

    def wait_in(t, h):
        pltpu.make_async_copy(wt_hbm.at[:, pl.ds(slab(t) * CHUNK, CHUNK)],
                              sbuf[h], isem[h]).wait()

    def out_desc(t, h):
        return pltpu.make_async_copy(
            obuf[h], wrow_hbm.at[pl.ds(slab(t) * CHUNK * DIM, CHUNK * DIM)],
            osem[h])

    def transpose(h):
        @plsc.parallel_loop(0, CHUNK, 16, unroll=2)
        def _(b0):
            woff = b0 * DIM
            for d0 in range(0, DIM, 16):
                rows = lane + d0
                for j in range(16):
                    v = plsc.load_gather(sbuf[h], [rows, cdiag[j] + b0])
                    plsc.store_scatter(obuf[h], [wdiag[j] + (woff + d0)], v)

    @pl.when(slab(0) < NSLAB)
    def _():
        fire_in(0, 0)

    def step(t, h):
        @pl.when(jnp.logical_and(t >= 2, slab(t - 2) < NSLAB))
        def _():
            out_desc(t - 2, h).wait()

        @pl.when(slab(t) < NSLAB)
        def _():
            wait_in(t, h)

            @pl.when(slab(t + 1) < NSLAB)
            def _():
                fire_in(t + 1, 1 - h)

            transpose(h)
            out_desc(t, h).start()

    def body(u, carry):
        step(2 * u, 0)
        step(2 * u + 1, 1)
        return carry

    lax.fori_loop(0, SPW_CEIL // 2, body, 0)
    step(SPW_CEIL - 1, (SPW_CEIL - 1) % 2)

    @pl.when(slab(SPW_CEIL - 2) < NSLAB)
    def _():
        out_desc(SPW_CEIL - 2, (SPW_CEIL - 2) % 2).wait()

    @pl.when(slab(SPW_CEIL - 1) < NSLAB)
    def _():
        out_desc(SPW_CEIL - 1, (SPW_CEIL - 1) % 2).wait()


@functools.partial(
    pl.kernel,
    mesh=_mesh,
    out_type=jax.ShapeDtypeStruct((S_SEQ, TR, B_SEQ // CHUNK, TILE), jnp.float32),
    scratch_types=[
        pltpu.VMEM((PPW, CHUNK), jnp.int32),
        pltpu.VMEM((2, CHUNK, DIM), jnp.float32),
        pltpu.VMEM((2, TR * TILE), jnp.float32),
        pltpu.SemaphoreType.DMA((2,)),
        pltpu.SemaphoreType.DMA((2,)),
    ],
    compiler_params=pltpu.CompilerParams(use_tc_tiling_on_sc=False,
                                         needs_layout_passes=False),
)
def _emb_lookup(table_hbm, idx_hbm, raw_hbm, idx_all, stag, tbuf, gsem, ssem):
    wid = lax.axis_index("s") * NC + lax.axis_index("c")
    base_p = wid * PPW

    pltpu.sync_copy(idx_hbm.at[pl.ds(base_p, PPW)], idx_all)

    lane = lax.iota(jnp.int32, 16)
    cdiag = [jnp.where(lane + j > 15, lane + j - 16, lane + j)
             for j in range(16)]
    wdiag = [cdiag[j] * CHUNK + lane for j in range(16)]

    def fire(k, h):
        pltpu.async_copy(table_hbm.at[idx_all.at[k]], stag.at[h], gsem.at[h])

    def gwait(k, h):
        pltpu.make_async_copy(table_hbm.at[idx_all.at[k]], stag.at[h],
                              gsem.at[h]).wait()

    def transpose(h):
        @plsc.parallel_loop(0, CHUNK, 16, unroll=2)
        def _(b0):
            rows = lane + b0
            for d0 in range(0, DIM, 16):
                for j in range(16):
                    v = plsc.load_gather(stag.at[h], [rows, cdiag[j] + d0])
                    plsc.store_scatter(tbuf.at[h],
                                       [wdiag[j] + (d0 * CHUNK + b0)], v)

    def store_descs(k, h):
        p = base_p + k
        s = p // (B_SEQ // CHUNK)
        tc = lax.rem(p, B_SEQ // CHUNK)
        return [
            pltpu.make_async_copy(tbuf.at[h, pl.ds(tr * TILE, TILE)],
                                  raw_hbm.at[s, tr, tc], ssem.at[h])
            for tr in range(TR)
        ]

    fire(0, 0)

    def loop_body(t, carry):
        for h in (0, 1):
            k = 2 * t + h
            gwait(k, h)
            if h == 0:
                fire(k + 1, 1)
            else:
                @pl.when(t < PPW // 2 - 1)
                def _():
                    fire(k + 1, 0)

            @pl.when(t >= 1)
            def _():
                for d in store_descs(k - 2, h):
                    d.wait()

            transpose(h)
            for d in store_descs(k, h):
                d.start()
        return carry

    lax.fori_loop(0, PPW // 2, loop_body, 0)
    for h in (0, 1):
        for d in store_descs(PPW - 2 + h, h):
            d.wait()


def kernel(x, weight):
    wrow = _table_transpose(weight.T)
    tail = weight[NSLAB * CHUNK:].reshape(-1)
    wrow = lax.dynamic_update_slice(wrow, tail, (NSLAB * CHUNK * DIM,))
    xt = x.T.reshape(NPAIR, CHUNK).astype(jnp.int32)
    raw = _emb_lookup(wrow.reshape(NUM_EMB, DIM), xt)
    out = raw.reshape(S_SEQ, TR, B_SEQ // CHUNK, 8, CHUNK)
    out = out.transpose(2, 4, 0, 1, 3).reshape(B_SEQ, S_SEQ, DIM)
    return out


# speedup vs baseline: 2.9164x; 1.3839x over previous
import functools

import jax
import jax.numpy as jnp
from jax import lax
from jax.experimental import pallas as pl
from jax.experimental.pallas import tpu as pltpu
from jax.experimental.pallas import tpu_sc as plsc

NUM_EMB = 1000000
DIM = 64
B_SEQ = 16384
S_SEQ = 50
B_TOTAL = B_SEQ * S_SEQ

_info = plsc.get_sparse_core_info()
NC, NS = _info.num_cores, _info.num_subcores
NW = NC * NS
CHUNK = 128
NPAIR = B_TOTAL // CHUNK
PPW = NPAIR // NW
TR = DIM // 8
TILE = 8 * CHUNK

_mesh = plsc.VectorSubcoreMesh(core_axis_name="c", subcore_axis_name="s")

NSLAB = NUM_EMB // CHUNK
SPW_CEIL = NSLAB // NW + 1


@functools.partial(
    pl.kernel,
    mesh=_mesh,
    out_type=jax.ShapeDtypeStruct((NUM_EMB * DIM,), jnp.float32),
    scratch_types=[
        pltpu.VMEM((DIM, CHUNK), jnp.float32),
        pltpu.VMEM((DIM, CHUNK), jnp.float32),
        pltpu.VMEM((CHUNK * DIM,), jnp.float32),
        pltpu.VMEM((CHUNK * DIM,), jnp.float32),
        pltpu.SemaphoreType.DMA,
        pltpu.SemaphoreType.DMA,
        pltpu.SemaphoreType.DMA,
        pltpu.SemaphoreType.DMA,
    ],
    compiler_params=pltpu.CompilerParams(use_tc_tiling_on_sc=True,
                                         needs_layout_passes=False),
)
def _table_transpose(wt_hbm, wrow_hbm, sbuf0, sbuf1, obuf0, obuf1,
                     isem0, isem1, osem0, osem1):
    wid = lax.axis_index("s") * NC + lax.axis_index("c")
    sbuf = (sbuf0, sbuf1)
    obuf = (obuf0, obuf1)
    isem = (isem0, isem1)
    osem = (osem0, osem1)

    lane = lax.iota(jnp.int32, 16)
    cdiag = [jnp.where(lane + j > 15, lane + j - 16, lane + j)
             for j in range(16)]
    wdiag = [cdiag[j] * DIM + lane for j in range(16)]

    def slab(t):
        return wid + NW * t

    def fire_in(t, h):
        pltpu.async_copy(wt_hbm.at[:, pl.ds(slab(t) * CHUNK, CHUNK)],
                         sbuf[h], isem[h])

    def wait_in(t, h):
        pltpu.make_async_copy(wt_hbm.at[:, pl.ds(slab(t) * CHUNK, CHUNK)],
                              sbuf[h], isem[h]).wait()

    def out_desc(t, h):
        return pltpu.make_async_copy(
            obuf[h], wrow_hbm.at[pl.ds(slab(t) * CHUNK * DIM, CHUNK * DIM)],
            osem[h])

    def transpose(h):
        @plsc.parallel_loop(0, CHUNK, 16, unroll=4)
        def _(b0):
            woff = b0 * DIM
            for d0 in range(0, DIM, 16):
                rows = lane + d0
                for j in range(16):
                    v = plsc.load_gather(sbuf[h], [rows, cdiag[j] + b0])
                    plsc.store_scatter(obuf[h], [wdiag[j] + (woff + d0)], v)

    @pl.when(slab(0) < NSLAB)
    def _():
        fire_in(0, 0)

    def step(t, h):
        @pl.when(jnp.logical_and(t >= 2, slab(t - 2) < NSLAB))
        def _():
            out_desc(t - 2, h).wait()

        @pl.when(slab(t) < NSLAB)
        def _():
            wait_in(t, h)

            @pl.when(slab(t + 1) < NSLAB)
            def _():
                fire_in(t + 1, 1 - h)

            transpose(h)
            out_desc(t, h).start()

    def body(u, carry):
        step(2 * u, 0)
        step(2 * u + 1, 1)
        return carry

    lax.fori_loop(0, SPW_CEIL // 2, body, 0)
    step(SPW_CEIL - 1, (SPW_CEIL - 1) % 2)

    @pl.when(slab(SPW_CEIL - 2) < NSLAB)
    def _():
        out_desc(SPW_CEIL - 2, (SPW_CEIL - 2) % 2).wait()

    @pl.when(slab(SPW_CEIL - 1) < NSLAB)
    def _():
        out_desc(SPW_CEIL - 1, (SPW_CEIL - 1) % 2).wait()


@functools.partial(
    pl.kernel,
    mesh=_mesh,
    out_type=jax.ShapeDtypeStruct((S_SEQ, TR, B_SEQ // CHUNK, TILE), jnp.float32),
    scratch_types=[
        pltpu.VMEM((PPW, CHUNK), jnp.int32),
        pltpu.VMEM((2, CHUNK, DIM), jnp.float32),
        pltpu.VMEM((2, TR * TILE), jnp.float32),
        pltpu.SemaphoreType.DMA((2,)),
        pltpu.SemaphoreType.DMA((2,)),
    ],
    compiler_params=pltpu.CompilerParams(use_tc_tiling_on_sc=False,
                                         needs_layout_passes=False),
)
def _emb_lookup(table_hbm, idx_hbm, raw_hbm, idx_all, stag, tbuf, gsem, ssem):
    wid = lax.axis_index("s") * NC + lax.axis_index("c")
    base_p = wid * PPW

    pltpu.sync_copy(idx_hbm.at[pl.ds(base_p, PPW)], idx_all)

    lane = lax.iota(jnp.int32, 16)
    cdiag = [jnp.where(lane + j > 15, lane + j - 16, lane + j)
             for j in range(16)]
    wdiag = [cdiag[j] * CHUNK + lane for j in range(16)]

    def fire(k, h):
        pltpu.async_copy(table_hbm.at[idx_all.at[k]], stag.at[h], gsem.at[h])

    def gwait(k, h):
        pltpu.make_async_copy(table_hbm.at[idx_all.at[k]], stag.at[h],
                              gsem.at[h]).wait()

    def transpose(h):
        @plsc.parallel_loop(0, CHUNK, 16, unroll=4)
        def _(b0):
            rows = lane + b0
            for d0 in range(0, DIM, 16):
                for j in range(16):
                    v = plsc.load_gather(stag.at[h], [rows, cdiag[j] + d0])
                    plsc.store_scatter(tbuf.at[h],
                                       [wdiag[j] + (d0 * CHUNK + b0)], v)

    def store_descs(k, h):
        p = base_p + k
        s = p // (B_SEQ // CHUNK)
        tc = lax.rem(p, B_SEQ // CHUNK)
        return [
            pltpu.make_async_copy(tbuf.at[h, pl.ds(tr * TILE, TILE)],
                                  raw_hbm.at[s, tr, tc], ssem.at[h])
            for tr in range(TR)
        ]

    fire(0, 0)

    def loop_body(t, carry):
        for h in (0, 1):
            k = 2 * t + h
            gwait(k, h)
            if h == 0:
                fire(k + 1, 1)
            else:
                @pl.when(t < PPW // 2 - 1)
                def _():
                    fire(k + 1, 0)

            @pl.when(t >= 1)
            def _():
                for d in store_descs(k - 2, h):
                    d.wait()

            transpose(h)
            for d in store_descs(k, h):
                d.start()
        return carry

    lax.fori_loop(0, PPW // 2, loop_body, 0)
    for h in (0, 1):
        for d in store_descs(PPW - 2 + h, h):
            d.wait()


def kernel(x, weight):
    wrow = _table_transpose(weight.T)
    tail = weight[NSLAB * CHUNK:].reshape(-1)
    wrow = lax.dynamic_update_slice(wrow, tail, (NSLAB * CHUNK * DIM,))
    xt = x.T.reshape(NPAIR, CHUNK).astype(jnp.int32)
    raw = _emb_lookup(wrow.reshape(NUM_EMB, DIM), xt)
    out = raw.reshape(S_SEQ, TR, B_SEQ // CHUNK, 8, CHUNK)
    out = out.transpose(2, 4, 0, 1, 3).reshape(B_SEQ, S_SEQ, DIM)
    return out
